# Initial kernel scaffold; baseline (speedup 1.0000x reference)
import jax
import jax.numpy as jnp
from jax.experimental import pallas as pl


def kernel(x, W1, b1, W2, b2, W3, b3, codebooks):
    raise NotImplementedError("write your pallas kernel here")


# speedup vs baseline: 4.9375x; 4.9375x over previous
import functools

import jax
import jax.numpy as jnp
from jax import lax
from jax.experimental import pallas as pl
from jax.experimental.pallas import tpu as pltpu
from jax.experimental.pallas import tpu_sc as plsc

B = 4096
D_IN = 1024
H1 = 2048
H2 = 1024
D_T = 512
S = 16
K = 1024
DV = D_T // S

TB = 256
TB2 = 512

_SQRT_HALF = 0.7071067811865476


def _gelu_exact(x):
    return 0.5 * x * (1.0 + lax.erf(x * _SQRT_HALF))


def _mlp_body(x_ref, w1_ref, b1_ref, w2_ref, b2_ref, w3_ref, b3_ref, enc_ref):
    f32 = jnp.float32
    h = jnp.dot(x_ref[...], w1_ref[...], preferred_element_type=f32)
    h = _gelu_exact(h + b1_ref[...])
    h = jnp.dot(h, w2_ref[...], preferred_element_type=f32)
    h = _gelu_exact(h + b2_ref[...])
    enc = jnp.dot(h, w3_ref[...], preferred_element_type=f32) + b3_ref[...]
    for s in range(S):
        enc_ref[s] = enc[:, s * DV:(s + 1) * DV]


def _mlp(x, W1, b1, W2, b2, W3, b3):
    rep = lambda *shape: pl.BlockSpec(shape, lambda i: (0,) * len(shape))
    return pl.pallas_call(
        _mlp_body,
        grid=(B // TB,),
        in_specs=[
            pl.BlockSpec((TB, D_IN), lambda i: (i, 0)),
            rep(D_IN, H1),
            rep(1, H1),
            rep(H1, H2),
            rep(1, H2),
            rep(H2, D_T),
            rep(1, D_T),
        ],
        out_specs=pl.BlockSpec((S, TB, DV), lambda i: (0, i, 0)),
        out_shape=jax.ShapeDtypeStruct((S, B, DV), jnp.float32),
    )(x, W1, b1.reshape(1, H1), W2, b2.reshape(1, H2), W3, b3.reshape(1, D_T))


def _codes_body(enc_ref, cbt_ref, codes_ref):
    s = pl.program_id(1)
    f32 = jnp.float32
    es = enc_ref[0]
    cbt = cbt_ref[0]
    cbn = jnp.sum(cbt * cbt, axis=0, keepdims=True)
    d = cbn + jnp.dot(es, -2.0 * cbt, preferred_element_type=f32)
    m = jnp.min(d, axis=1, keepdims=True)
    iota = lax.broadcasted_iota(jnp.int32, (TB2, K), 1)
    code = jnp.min(jnp.where(d <= m, iota, K), axis=1)
    codes_ref[0, 0] = code + s * K


def _codes(enc_t, cbt):
    return pl.pallas_call(
        _codes_body,
        grid=(B // TB2, S),
        in_specs=[
            pl.BlockSpec((1, TB2, DV), lambda i, s: (s, i, 0)),
            pl.BlockSpec((1, DV, K), lambda i, s: (s, 0, 0)),
        ],
        out_specs=pl.BlockSpec((1, 1, TB2), lambda i, s: (s, 0, i)),
        out_shape=jax.ShapeDtypeStruct((S, 1, B), jnp.int32),
    )(enc_t, cbt)


_NC, _NS = 2, 16
_NW = _NC * _NS
_TOTAL = B * S
_PER_W = _TOTAL // _NW
_CHUNK = 128
_NCH = _PER_W // _CHUNK


@functools.cache
def _make_sc_gather():
    @functools.partial(
        pl.kernel,
        out_type=jax.ShapeDtypeStruct((_TOTAL, DV), jnp.float32),
        mesh=plsc.VectorSubcoreMesh(core_axis_name="c", subcore_axis_name="s"),
        scratch_types=[
            pltpu.VMEM((_NCH, _CHUNK), jnp.int32),
            pltpu.VMEM((_PER_W, DV), jnp.float32),
            pltpu.SemaphoreType.DMA,
        ],
        compiler_params=pltpu.CompilerParams(use_tc_tiling_on_sc=False),
    )
    def _sc_gather(idx_hbm, table_hbm, out_hbm, idx_v, rows_v, sem):
        wid = lax.axis_index("s") * _NC + lax.axis_index("c")
        pltpu.sync_copy(idx_hbm.at[wid], idx_v)
        copies = []
        for j in range(_NCH):
            copies.append(pltpu.async_copy(
                table_hbm.at[idx_v.at[j]],
                rows_v.at[pl.ds(j * _CHUNK, _CHUNK)],
                sem))
        for c in copies:
            c.wait()
        pltpu.sync_copy(rows_v, out_hbm.at[pl.ds(wid * _PER_W, _PER_W)])

    return _sc_gather


def kernel(x, W1, b1, W2, b2, W3, b3, codebooks):
    enc_t = _mlp(x, W1, b1, W2, b2, W3, b3)
    cbt = codebooks.transpose(0, 2, 1)
    codes_t = _codes(enc_t, cbt)
    idx = codes_t.reshape(S, B).T.reshape(_NW, _NCH, _CHUNK)
    table = codebooks.reshape(S * K, DV)
    rows = _make_sc_gather()(idx, table)
    return rows.reshape(B, D_T)


# speedup vs baseline: 7.2984x; 1.4782x over previous
import functools

import jax
import jax.numpy as jnp
from jax import lax
from jax.experimental import pallas as pl
from jax.experimental.pallas import tpu as pltpu
from jax.experimental.pallas import tpu_sc as plsc

B = 4096
D_IN = 1024
H1 = 2048
H2 = 1024
D_T = 512
S = 16
K = 1024
DV = D_T // S

TB = 1024
TB2 = 4096

_SQRT_HALF = 0.7071067811865476


def _gelu_exact(x):
    return 0.5 * x * (1.0 + lax.erf(x * _SQRT_HALF))


def _mlp_body(x_ref, w1_ref, b1_ref, w2_ref, b2_ref, w3_ref, b3_ref, enc_ref):
    f32 = jnp.float32
    h = jnp.dot(x_ref[...], w1_ref[...], preferred_element_type=f32)
    h = _gelu_exact(h + b1_ref[...])
    h = jnp.dot(h, w2_ref[...], preferred_element_type=f32)
    h = _gelu_exact(h + b2_ref[...])
    enc = jnp.dot(h, w3_ref[...], preferred_element_type=f32) + b3_ref[...]
    for s in range(S):
        enc_ref[s] = enc[:, s * DV:(s + 1) * DV]


def _mlp(x, W1, b1, W2, b2, W3, b3):
    rep = lambda *shape: pl.BlockSpec(shape, lambda i: (0,) * len(shape))
    return pl.pallas_call(
        _mlp_body,
        grid=(B // TB,),
        in_specs=[
            pl.BlockSpec((TB, D_IN), lambda i: (i, 0)),
            rep(D_IN, H1),
            rep(1, H1),
            rep(H1, H2),
            rep(1, H2),
            rep(H2, D_T),
            rep(1, D_T),
        ],
        out_specs=pl.BlockSpec((S, TB, DV), lambda i: (0, i, 0)),
        out_shape=jax.ShapeDtypeStruct((S, B, DV), jnp.float32),
    )(x, W1, b1.reshape(1, H1), W2, b2.reshape(1, H2), W3, b3.reshape(1, D_T))


def _codes_body(enc_ref, cbt_ref, codes_ref):
    s = pl.program_id(1)
    f32 = jnp.float32
    es = enc_ref[0]
    cbt = cbt_ref[0]
    cbn = jnp.sum(cbt * cbt, axis=0, keepdims=True)
    d = cbn + jnp.dot(es, -2.0 * cbt, preferred_element_type=f32)
    code = jnp.argmin(d, axis=1).astype(jnp.int32)
    codes_ref[0] = (code + s * K)[:, None]


def _codes(enc_t, cbt):
    return pl.pallas_call(
        _codes_body,
        grid=(B // TB2, S),
        in_specs=[
            pl.BlockSpec((1, TB2, DV), lambda i, s: (s, i, 0)),
            pl.BlockSpec((1, DV, K), lambda i, s: (s, 0, 0)),
        ],
        out_specs=pl.BlockSpec((1, TB2, 1), lambda i, s: (s, i, 0)),
        out_shape=jax.ShapeDtypeStruct((S, B, 1), jnp.int32),
    )(enc_t, cbt)


_NC, _NS = 2, 16
_NW = _NC * _NS
_TOTAL = B * S
_PER_W = _TOTAL // _NW
_CHUNK = 128
_NCH = _PER_W // _CHUNK


@functools.cache
def _make_sc_gather():
    @functools.partial(
        pl.kernel,
        out_type=jax.ShapeDtypeStruct((_TOTAL, DV), jnp.float32),
        mesh=plsc.VectorSubcoreMesh(core_axis_name="c", subcore_axis_name="s"),
        scratch_types=[
            pltpu.VMEM((_NCH, _CHUNK), jnp.int32),
            pltpu.VMEM((_PER_W, DV), jnp.float32),
            pltpu.SemaphoreType.DMA,
        ],
        compiler_params=pltpu.CompilerParams(use_tc_tiling_on_sc=False),
    )
    def _sc_gather(idx_hbm, table_hbm, out_hbm, idx_v, rows_v, sem):
        wid = lax.axis_index("s") * _NC + lax.axis_index("c")
        pltpu.sync_copy(idx_hbm.at[wid], idx_v)
        copies = []
        for j in range(_NCH):
            copies.append(pltpu.async_copy(
                table_hbm.at[idx_v.at[j]],
                rows_v.at[pl.ds(j * _CHUNK, _CHUNK)],
                sem))
        for c in copies:
            c.wait()
        pltpu.sync_copy(rows_v, out_hbm.at[pl.ds(wid * _PER_W, _PER_W)])

    return _sc_gather


def kernel(x, W1, b1, W2, b2, W3, b3, codebooks):
    enc_t = _mlp(x, W1, b1, W2, b2, W3, b3)
    cbt = codebooks.transpose(0, 2, 1)
    codes_t = _codes(enc_t, cbt)
    idx = codes_t.reshape(S, B).T.reshape(_NW, _NCH, _CHUNK)
    table = codebooks.reshape(S * K, DV)
    rows = _make_sc_gather()(idx, table)
    return rows.reshape(B, D_T)


# speedup vs baseline: 7.6705x; 1.0510x over previous
import functools

import jax
import jax.numpy as jnp
from jax import lax
from jax.experimental import pallas as pl
from jax.experimental.pallas import tpu as pltpu
from jax.experimental.pallas import tpu_sc as plsc

B = 4096
D_IN = 1024
H1 = 2048
H2 = 1024
D_T = 512
S = 16
K = 1024
DV = D_T // S

TB = 1024
TB2 = 4096

_SQRT_HALF = 0.7071067811865476


def _gelu_exact(x):
    return 0.5 * x * (1.0 + lax.erf(x * _SQRT_HALF))


def _fused_body(x_ref, w1_ref, b1_ref, w2_ref, b2_ref, w3_ref, b3_ref,
                cbt_ref, codes_ref):
    f32 = jnp.float32
    h = jnp.dot(x_ref[...], w1_ref[...], preferred_element_type=f32)
    h = _gelu_exact(h + b1_ref[...])
    h = jnp.dot(h, w2_ref[...], preferred_element_type=f32)
    h = _gelu_exact(h + b2_ref[...])
    enc = jnp.dot(h, w3_ref[...], preferred_element_type=f32) + b3_ref[...]
    for s in range(S):
        es = enc[:, s * DV:(s + 1) * DV]
        cbt = cbt_ref[s]
        cbn = jnp.sum(cbt * cbt, axis=0, keepdims=True)
        d = cbn + jnp.dot(es, -2.0 * cbt, preferred_element_type=f32)
        code = jnp.argmin(d, axis=1).astype(jnp.int32)
        codes_ref[s] = (code + s * K)[:, None]


def _fused(x, W1, b1, W2, b2, W3, b3, cbt):
    rep = lambda *shape: pl.BlockSpec(shape, lambda i: (0,) * len(shape))
    return pl.pallas_call(
        _fused_body,
        grid=(B // TB,),
        in_specs=[
            pl.BlockSpec((TB, D_IN), lambda i: (i, 0)),
            rep(D_IN, H1),
            rep(1, H1),
            rep(H1, H2),
            rep(1, H2),
            rep(H2, D_T),
            rep(1, D_T),
            rep(S, DV, K),
        ],
        out_specs=pl.BlockSpec((S, TB, 1), lambda i: (0, i, 0)),
        out_shape=jax.ShapeDtypeStruct((S, B, 1), jnp.int32),
    )(x, W1, b1.reshape(1, H1), W2, b2.reshape(1, H2), W3, b3.reshape(1, D_T),
      cbt)


_NC, _NS = 2, 16
_NW = _NC * _NS
_TOTAL = B * S
_PER_W = _TOTAL // _NW
_CHUNK = 128
_NCH = _PER_W // _CHUNK


@functools.cache
def _make_sc_gather():
    @functools.partial(
        pl.kernel,
        out_type=jax.ShapeDtypeStruct((_TOTAL, DV), jnp.float32),
        mesh=plsc.VectorSubcoreMesh(core_axis_name="c", subcore_axis_name="s"),
        scratch_types=[
            pltpu.VMEM((_NCH, _CHUNK), jnp.int32),
            pltpu.VMEM((_PER_W, DV), jnp.float32),
            pltpu.SemaphoreType.DMA,
        ],
        compiler_params=pltpu.CompilerParams(use_tc_tiling_on_sc=False),
    )
    def _sc_gather(idx_hbm, table_hbm, out_hbm, idx_v, rows_v, sem):
        wid = lax.axis_index("s") * _NC + lax.axis_index("c")
        pltpu.sync_copy(idx_hbm.at[wid], idx_v)
        copies = []
        for j in range(_NCH):
            copies.append(pltpu.async_copy(
                table_hbm.at[idx_v.at[j]],
                rows_v.at[pl.ds(j * _CHUNK, _CHUNK)],
                sem))
        for c in copies:
            c.wait()
        pltpu.sync_copy(rows_v, out_hbm.at[pl.ds(wid * _PER_W, _PER_W)])

    return _sc_gather


def kernel(x, W1, b1, W2, b2, W3, b3, codebooks):
    cbt = codebooks.transpose(0, 2, 1)
    codes_t = _fused(x, W1, b1, W2, b2, W3, b3, cbt)
    idx = codes_t.reshape(S, B).T.reshape(_NW, _NCH, _CHUNK)
    table = codebooks.reshape(S * K, DV)
    rows = _make_sc_gather()(idx, table)
    return rows.reshape(B, D_T)
